# baseline (speedup 1.0000x reference)
import functools

import jax
import jax.numpy as jnp
from jax import lax
from jax.experimental import pallas as pl
from jax.experimental.pallas import tpu as pltpu
from jax.experimental.pallas import tpu_sc as plsc

NC = 2
NS = 16
NW = NC * NS


def _sc_mesh():
    return plsc.VectorSubcoreMesh(core_axis_name="c", subcore_axis_name="s")


def _wid():
    return lax.axis_index("s") * NC + lax.axis_index("c")


def _make_sc_gather(T, D, R, chunks, chunk_rows):
    @functools.partial(
        pl.kernel,
        mesh=_sc_mesh(),
        out_type=jax.ShapeDtypeStruct((R, D), jnp.float32),
        scratch_types=[
            pltpu.VMEM((chunks, chunk_rows), jnp.int32),
            pltpu.VMEM((chunk_rows, D), jnp.float32),
            pltpu.VMEM((chunk_rows, D), jnp.float32),
            pltpu.SemaphoreType.DMA,
            pltpu.SemaphoreType.DMA,
        ],
    )
    def sc_gather(x_hbm, idx_hbm, xs_hbm, idx_v, buf0, buf1, sem0, sem1):
        wid = _wid()
        base = wid * (chunks * chunk_rows)
        pltpu.sync_copy(idx_hbm.at[wid], idx_v)
        bufs = (buf0, buf1)
        sems = (sem0, sem1)
        cps = [None, None]
        for c in range(chunks):
            s = c % 2
            cps[s] = pltpu.async_copy(x_hbm.at[idx_v.at[c]], bufs[s], sems[s])
            if c >= 1:
                p = (c - 1) % 2
                cps[p].wait()
                pltpu.sync_copy(
                    bufs[p], xs_hbm.at[pl.ds(base + (c - 1) * chunk_rows, chunk_rows)]
                )
        last = (chunks - 1) % 2
        cps[last].wait()
        pltpu.sync_copy(
            bufs[last], xs_hbm.at[pl.ds(base + (chunks - 1) * chunk_rows, chunk_rows)]
        )

    return sc_gather


def _make_sc_combine(T, D, K, R, tpw):
    assert K == 2

    @functools.partial(
        pl.kernel,
        mesh=_sc_mesh(),
        out_type=jax.ShapeDtypeStruct((T, D), jnp.float32),
        scratch_types=[
            pltpu.VMEM((K, tpw), jnp.int32),
            pltpu.VMEM((tpw, D), jnp.float32),
            pltpu.VMEM((tpw, D), jnp.float32),
            pltpu.SemaphoreType.DMA,
            pltpu.SemaphoreType.DMA,
        ],
    )
    def sc_combine(outw_hbm, inv_hbm, y_hbm, idx_v, bufa, bufb, sema, semb):
        wid = _wid()
        pltpu.sync_copy(inv_hbm.at[wid], idx_v)
        ca = pltpu.async_copy(outw_hbm.at[idx_v.at[0]], bufa, sema)
        cb = pltpu.async_copy(outw_hbm.at[idx_v.at[1]], bufb, semb)
        ca.wait()
        cb.wait()

        def row_add(i, carry):
            for j in range(D // 16):
                sl = pl.ds(j * 16, 16)
                bufa[i, sl] = bufa[i, sl] + bufb[i, sl]
            return carry

        lax.fori_loop(0, tpw, row_add, 0)
        pltpu.sync_copy(bufa, y_hbm.at[pl.ds(wid * tpw, tpw)])

    return sc_combine


def _ffn_body(bexp_ref, xs_ref, ws_ref, wg_ref, wu_ref, wd_ref, out_ref):
    del bexp_ref
    bm = xs_ref.shape[0]
    X = xs_ref[...]
    g = jnp.dot(X, wg_ref[0], preferred_element_type=jnp.float32)
    u = jnp.dot(X, wu_ref[0], preferred_element_type=jnp.float32)
    h = g * jax.nn.sigmoid(g) * u
    i0 = lax.broadcasted_iota(jnp.int32, (bm, bm), 0)
    i1 = lax.broadcasted_iota(jnp.int32, (bm, bm), 1)
    diag = jnp.where(i0 == i1, jnp.broadcast_to(ws_ref[0], (bm, bm)), 0.0)
    hw = jnp.dot(diag, h, preferred_element_type=jnp.float32)
    out_ref[...] = jnp.dot(hw, wd_ref[0], preferred_element_type=jnp.float32)


def kernel(x, token_to_expert_indices, weights, Wg, Wu, Wd):
    T, D = x.shape
    E, _, H = Wg.shape
    K = token_to_expert_indices.shape[1]
    S = T * K
    BM = 128
    NB = S // BM + E
    R = NB * BM

    e_flat = token_to_expert_indices.astype(jnp.int32).reshape(S)
    oh = (e_flat[:, None] == jnp.arange(E, dtype=jnp.int32)[None, :]).astype(jnp.int32)
    rank = jnp.sum((jnp.cumsum(oh, axis=0) - 1) * oh, axis=1)
    counts = jnp.sum(oh, axis=0)
    padded = ((counts + BM - 1) // BM) * BM
    poff = jnp.cumsum(padded) - padded
    pos = poff[e_flat] + rank
    tok = jnp.arange(S, dtype=jnp.int32) // K
    row_src = jnp.zeros(R, jnp.int32).at[pos].set(tok)
    ws = jnp.zeros(R, jnp.float32).at[pos].set(
        weights.reshape(S).astype(jnp.float32))
    pend = poff + padded
    bstart = jnp.arange(NB, dtype=jnp.int32) * BM
    block_expert = jnp.minimum(
        jnp.sum((bstart[:, None] >= pend[None, :]).astype(jnp.int32), axis=1), E - 1
    ).astype(jnp.int32)

    chunk_rows = 64
    chunks = R // (NW * chunk_rows)
    idx_g = row_src.reshape(NW, chunks, chunk_rows)

    tpw = T // NW
    inv = pos.reshape(NW, tpw, K).transpose(0, 2, 1)

    xs = _make_sc_gather(T, D, R, chunks, chunk_rows)(x, idx_g)

    grid_spec = pltpu.PrefetchScalarGridSpec(
        num_scalar_prefetch=1,
        grid=(NB,),
        in_specs=[
            pl.BlockSpec((BM, D), lambda b, bexp: (b, 0)),
            pl.BlockSpec((1, 1, BM), lambda b, bexp: (b, 0, 0)),
            pl.BlockSpec((1, D, H), lambda b, bexp: (bexp[b], 0, 0)),
            pl.BlockSpec((1, D, H), lambda b, bexp: (bexp[b], 0, 0)),
            pl.BlockSpec((1, H, D), lambda b, bexp: (bexp[b], 0, 0)),
        ],
        out_specs=pl.BlockSpec((BM, D), lambda b, bexp: (b, 0)),
    )
    outw = pl.pallas_call(
        _ffn_body,
        grid_spec=grid_spec,
        out_shape=jax.ShapeDtypeStruct((R, D), jnp.float32),
    )(block_expert, xs, ws.reshape(NB, 1, BM), Wg, Wu, Wd)

    y = _make_sc_combine(T, D, K, R, tpw)(outw, inv)
    return y


# speedup vs baseline: 1.9755x; 1.9755x over previous
import functools

import jax
import jax.numpy as jnp
from jax import lax
from jax.experimental import pallas as pl
from jax.experimental.pallas import tpu as pltpu
from jax.experimental.pallas import tpu_sc as plsc

NC = 2
NS = 16
NW = NC * NS


def _sc_mesh():
    return plsc.VectorSubcoreMesh(core_axis_name="c", subcore_axis_name="s")


def _wid():
    return lax.axis_index("s") * NC + lax.axis_index("c")


def _make_sc_dispatch(T, D, K, R, tpw):
    @functools.partial(
        pl.kernel,
        mesh=_sc_mesh(),
        out_type=jax.ShapeDtypeStruct((R, D), jnp.float32),
        scratch_types=[
            pltpu.VMEM((K, tpw), jnp.int32),
            pltpu.VMEM((tpw, D), jnp.float32),
            pltpu.SemaphoreType.DMA,
        ],
    )
    def sc_dispatch(x_hbm, pos_hbm, xs_hbm, idx_v, buf, sem):
        wid = _wid()
        pltpu.sync_copy(pos_hbm.at[wid], idx_v)
        pltpu.sync_copy(x_hbm.at[pl.ds(wid * tpw, tpw)], buf)
        cps = [
            pltpu.async_copy(buf, xs_hbm.at[idx_v.at[k]], sem) for k in range(K)
        ]
        for cp in cps:
            cp.wait()

    return sc_dispatch


def _make_sc_combine(T, D, K, R, tpw):
    assert K == 2

    @functools.partial(
        pl.kernel,
        mesh=_sc_mesh(),
        out_type=jax.ShapeDtypeStruct((T, D), jnp.float32),
        scratch_types=[
            pltpu.VMEM((K, tpw), jnp.int32),
            pltpu.VMEM((K, tpw + 16), jnp.float32),
            pltpu.VMEM((tpw, D), jnp.float32),
            pltpu.VMEM((tpw, D), jnp.float32),
            pltpu.SemaphoreType.DMA,
            pltpu.SemaphoreType.DMA,
        ],
    )
    def sc_combine(outw_hbm, pos_hbm, w_hbm, y_hbm, idx_v, w_v, bufa, bufb,
                   sema, semb):
        wid = _wid()
        pltpu.sync_copy(pos_hbm.at[wid], idx_v)
        pltpu.sync_copy(w_hbm.at[wid], w_v)
        ca = pltpu.async_copy(outw_hbm.at[idx_v.at[0]], bufa, sema)
        cb = pltpu.async_copy(outw_hbm.at[idx_v.at[1]], bufb, semb)
        ca.wait()
        cb.wait()

        def row_fma(i, carry):
            wa = w_v[0, pl.ds(i, 16)][0]
            wb = w_v[1, pl.ds(i, 16)][0]
            for j in range(D // 16):
                sl = pl.ds(j * 16, 16)
                bufa[i, sl] = wa * bufa[i, sl] + wb * bufb[i, sl]
            return carry

        lax.fori_loop(0, tpw, row_fma, 0)
        pltpu.sync_copy(bufa, y_hbm.at[pl.ds(wid * tpw, tpw)])

    return sc_combine


def _ffn_body(bexp_ref, xs_ref, wg_ref, wu_ref, wd_ref, out_ref):
    del bexp_ref
    X = xs_ref[...]
    g = jnp.dot(X, wg_ref[0], preferred_element_type=jnp.float32)
    u = jnp.dot(X, wu_ref[0], preferred_element_type=jnp.float32)
    h = g * jax.nn.sigmoid(g) * u
    out_ref[...] = jnp.dot(h, wd_ref[0], preferred_element_type=jnp.float32)


def kernel(x, token_to_expert_indices, weights, Wg, Wu, Wd):
    T, D = x.shape
    E, _, H = Wg.shape
    K = token_to_expert_indices.shape[1]
    S = T * K
    BM = 128
    NB = S // BM + E
    R = NB * BM
    tpw = T // NW

    e_flat = token_to_expert_indices.astype(jnp.int32).reshape(S)
    oh = (e_flat[:, None] == jnp.arange(E, dtype=jnp.int32)[None, :]).astype(jnp.int32)
    rank = jnp.sum((jnp.cumsum(oh, axis=0) - 1) * oh, axis=1)
    counts = jnp.sum(oh, axis=0)
    padded = ((counts + BM - 1) // BM) * BM
    poff = jnp.cumsum(padded) - padded
    pos = poff[e_flat] + rank
    pend = poff + padded
    bstart = jnp.arange(NB, dtype=jnp.int32) * BM
    block_expert = jnp.minimum(
        jnp.sum((bstart[:, None] >= pend[None, :]).astype(jnp.int32), axis=1), E - 1
    ).astype(jnp.int32)

    pos3 = pos.reshape(NW, tpw, K).transpose(0, 2, 1)
    w3 = jnp.pad(
        weights.astype(jnp.float32).reshape(NW, tpw, K).transpose(0, 2, 1),
        ((0, 0), (0, 0), (0, 16)))

    xs = _make_sc_dispatch(T, D, K, R, tpw)(x, pos3)

    grid_spec = pltpu.PrefetchScalarGridSpec(
        num_scalar_prefetch=1,
        grid=(NB,),
        in_specs=[
            pl.BlockSpec((BM, D), lambda b, bexp: (b, 0)),
            pl.BlockSpec((1, D, H), lambda b, bexp: (bexp[b], 0, 0)),
            pl.BlockSpec((1, D, H), lambda b, bexp: (bexp[b], 0, 0)),
            pl.BlockSpec((1, H, D), lambda b, bexp: (bexp[b], 0, 0)),
        ],
        out_specs=pl.BlockSpec((BM, D), lambda b, bexp: (b, 0)),
    )
    outw = pl.pallas_call(
        _ffn_body,
        grid_spec=grid_spec,
        out_shape=jax.ShapeDtypeStruct((R, D), jnp.float32),
    )(block_expert, xs, Wg, Wu, Wd)

    y = _make_sc_combine(T, D, K, R, tpw)(outw, pos3, w3)
    return y
